# Initial kernel scaffold; baseline (speedup 1.0000x reference)
import jax
import jax.numpy as jnp
from jax.experimental import pallas as pl


def kernel(x, edge_index, W1, b1, W2, b2):
    raise NotImplementedError("write your pallas kernel here")


# speedup vs baseline: 6.7610x; 6.7610x over previous
import functools

_INTERPRET = False

import jax
import jax.numpy as jnp
from jax import lax
from jax.experimental import pallas as pl
from jax.experimental.pallas import tpu as pltpu
from jax.experimental.pallas import tpu_sc as plsc

N_NODES = 10000
D = 128

NC = 2
NS = 16
NW = NC * NS
N_PAD = 10240
ROWS_PER_TILE = N_PAD // NS
CHUNK = 128
E_PAD = 327680
CHUNKS_PER_W = E_PAD // (NW * CHUNK)
EDGES_PER_W = E_PAD // NW


DEGW = 128


def _sc_degree_body(dst_hbm, out_hbm, didx, ones_rows, zbuf, acc):
    c = lax.axis_index("c")
    s = lax.axis_index("s")
    wid = c * NS + s

    ones16 = jnp.ones((16,), jnp.float32)
    zeros16 = jnp.zeros((16,), jnp.float32)

    nq = DEGW // 16

    def fill_bufs(i, _):
        ones_rows[i // nq, pl.ds((i % nq) * 16, 16)] = ones16
        return 0

    lax.fori_loop(0, CHUNK * nq, fill_bufs, 0)

    def fill_z(i, _):
        zbuf[i // nq, pl.ds((i % nq) * 16, 16)] = zeros16
        return 0

    lax.fori_loop(0, 16 * nq, fill_z, 0)

    row0 = s * ROWS_PER_TILE

    def zero_acc(j, _):
        pltpu.sync_copy(zbuf, acc.at[pl.ds(row0 + j * 16, 16)])
        return 0

    lax.fori_loop(0, ROWS_PER_TILE // 16, zero_acc, 0)

    plsc.subcore_barrier()

    ebase = wid * EDGES_PER_W

    def chunk_body(j, _):
        pltpu.sync_copy(dst_hbm.at[pl.ds(ebase + j * CHUNK, CHUNK)], didx)
        pltpu.sync_copy(ones_rows, acc.at[didx], add=True)
        return 0

    lax.fori_loop(0, CHUNKS_PER_W, chunk_body, 0)

    plsc.subcore_barrier()

    pltpu.sync_copy(
        acc.at[pl.ds(row0, ROWS_PER_TILE)],
        out_hbm.at[c, pl.ds(row0, ROWS_PER_TILE)],
    )


def _sc_degree(dst_pad):
    mesh = plsc.VectorSubcoreMesh(core_axis_name="c", subcore_axis_name="s", num_cores=NC, num_subcores=NS)
    return pl.kernel(
        _sc_degree_body,
        out_type=jax.ShapeDtypeStruct((NC, N_PAD, DEGW), jnp.float32),
        mesh=mesh,
        interpret=_INTERPRET,
        scratch_types=[
            pltpu.VMEM((CHUNK,), jnp.int32),
            pltpu.VMEM((CHUNK, DEGW), jnp.float32),
            pltpu.VMEM((16, DEGW), jnp.float32),
            pltpu.VMEM_SHARED((N_PAD, DEGW), jnp.float32),
        ],
    )(dst_pad)


def _sc_agg_body(g_hbm, src_hbm, dst_hbm, out_hbm, sidx, didx, rows, zbuf, acc):
    c = lax.axis_index("c")
    s = lax.axis_index("s")
    wid = c * NS + s

    zeros16 = jnp.zeros((16,), jnp.float32)

    def zero_zbuf(i, _):
        r = i // 8
        q = i % 8
        zbuf[r, pl.ds(q * 16, 16)] = zeros16
        return 0

    lax.fori_loop(0, 16 * 8, zero_zbuf, 0)

    row0 = s * ROWS_PER_TILE

    def zero_acc(j, _):
        pltpu.sync_copy(zbuf, acc.at[pl.ds(row0 + j * 16, 16)])
        return 0

    lax.fori_loop(0, ROWS_PER_TILE // 16, zero_acc, 0)

    plsc.subcore_barrier()

    ebase = wid * EDGES_PER_W

    def chunk_body(j, _):
        off = ebase + j * CHUNK
        pltpu.sync_copy(src_hbm.at[pl.ds(off, CHUNK)], sidx)
        pltpu.sync_copy(dst_hbm.at[pl.ds(off, CHUNK)], didx)
        pltpu.sync_copy(g_hbm.at[sidx], rows)
        pltpu.sync_copy(rows, acc.at[didx], add=True)
        return 0

    lax.fori_loop(0, CHUNKS_PER_W, chunk_body, 0)

    plsc.subcore_barrier()

    pltpu.sync_copy(
        acc.at[pl.ds(row0, ROWS_PER_TILE)],
        out_hbm.at[c, pl.ds(row0, ROWS_PER_TILE)],
    )


def _sc_agg(g, src_pad, dst_pad):
    mesh = plsc.VectorSubcoreMesh(core_axis_name="c", subcore_axis_name="s", num_cores=NC, num_subcores=NS)
    return pl.kernel(
        _sc_agg_body,
        out_type=jax.ShapeDtypeStruct((NC, N_PAD, D), jnp.float32),
        mesh=mesh,
        interpret=_INTERPRET,
        scratch_types=[
            pltpu.VMEM((CHUNK,), jnp.int32),
            pltpu.VMEM((CHUNK,), jnp.int32),
            pltpu.VMEM((CHUNK, D), jnp.float32),
            pltpu.VMEM((16, D), jnp.float32),
            pltpu.VMEM_SHARED((N_PAD, D), jnp.float32),
        ],
    )(g, src_pad, dst_pad)


BLK = 400


def _tc_pre_body(x_ref, w_ref, degp_ref, g_ref, dinv_ref):
    i = pl.program_id(0)
    degs = degp_ref[0, pl.ds(i * BLK, BLK), :] + degp_ref[1, pl.ds(i * BLK, BLK), :]
    deg = degs[:, 0:1] + 1.0
    dinv = lax.rsqrt(deg)
    h = jnp.dot(x_ref[...], w_ref[...], preferred_element_type=jnp.float32)
    g_ref[...] = h * dinv
    dinv_ref[...] = dinv


def _tc_pre(x, W1, degp):
    return pl.pallas_call(
        _tc_pre_body,
        grid=(N_NODES // BLK,),
        interpret=_INTERPRET,
        in_specs=[
            pl.BlockSpec((BLK, D), lambda i: (i, 0)),
            pl.BlockSpec((D, D), lambda i: (0, 0)),
            pl.BlockSpec((NC, N_PAD, DEGW), lambda i: (0, 0, 0)),
        ],
        out_specs=[
            pl.BlockSpec((BLK, D), lambda i: (i, 0)),
            pl.BlockSpec((BLK, 1), lambda i: (i, 0)),
        ],
        out_shape=[
            jax.ShapeDtypeStruct((N_NODES, D), jnp.float32),
            jax.ShapeDtypeStruct((N_NODES, 1), jnp.float32),
        ],
    )(x, W1, degp)


def _tc_mid_body(p_ref, g1_ref, dinv_ref, b1_ref, w2_ref, g2_ref):
    s = p_ref[0] + p_ref[1] + g1_ref[...]
    dinv = dinv_ref[...]
    h = s * dinv + b1_ref[...]
    a = jnp.maximum(h, 0.0)
    h2 = jnp.dot(a, w2_ref[...], preferred_element_type=jnp.float32)
    g2_ref[...] = h2 * dinv


def _tc_mid(p, g1, dinv, b1, W2):
    return pl.pallas_call(
        _tc_mid_body,
        grid=(N_NODES // BLK,),
        interpret=_INTERPRET,
        in_specs=[
            pl.BlockSpec((NC, BLK, D), lambda i: (0, i, 0)),
            pl.BlockSpec((BLK, D), lambda i: (i, 0)),
            pl.BlockSpec((BLK, 1), lambda i: (i, 0)),
            pl.BlockSpec((1, D), lambda i: (0, 0)),
            pl.BlockSpec((D, D), lambda i: (0, 0)),
        ],
        out_specs=pl.BlockSpec((BLK, D), lambda i: (i, 0)),
        out_shape=jax.ShapeDtypeStruct((N_NODES, D), jnp.float32),
    )(p, g1, dinv, b1, W2)


def _tc_final_body(q_ref, g2_ref, dinv_ref, b2_ref, out_ref):
    s = q_ref[0] + q_ref[1] + g2_ref[...]
    z = s * dinv_ref[...] + b2_ref[...]
    m = jnp.max(z, axis=1, keepdims=True)
    e = jnp.exp(z - m)
    lse = m + jnp.log(jnp.sum(e, axis=1, keepdims=True))
    out_ref[...] = z - lse


def _tc_final(q, g2, dinv, b2):
    return pl.pallas_call(
        _tc_final_body,
        grid=(N_NODES // BLK,),
        interpret=_INTERPRET,
        in_specs=[
            pl.BlockSpec((NC, BLK, D), lambda i: (0, i, 0)),
            pl.BlockSpec((BLK, D), lambda i: (i, 0)),
            pl.BlockSpec((BLK, 1), lambda i: (i, 0)),
            pl.BlockSpec((1, D), lambda i: (0, 0)),
        ],
        out_specs=pl.BlockSpec((BLK, D), lambda i: (i, 0)),
        out_shape=jax.ShapeDtypeStruct((N_NODES, D), jnp.float32),
    )(q, g2, dinv, b2)


@jax.jit
def _run(x, edge_index, W1, b1, W2, b2):
    src = edge_index[0].astype(jnp.int32)
    dst = edge_index[1].astype(jnp.int32)
    n_edges = src.shape[0]
    pad = E_PAD - n_edges
    src_p = jnp.concatenate([src, jnp.zeros((pad,), jnp.int32)])
    dst_p = jnp.concatenate([dst, jnp.full((pad,), N_NODES, jnp.int32)])

    degp = _sc_degree(dst_p)
    g1, dinv = _tc_pre(x, W1, degp)
    p = _sc_agg(g1, src_p, dst_p)
    g2 = _tc_mid(p, g1, dinv, b1.reshape(1, D), W2)
    q = _sc_agg(g2, src_p, dst_p)
    return _tc_final(q, g2, dinv, b2.reshape(1, D))


def kernel(x, edge_index, W1, b1, W2, b2):
    return _run(x, edge_index, W1, b1, W2, b2)


# speedup vs baseline: 8.3999x; 1.2424x over previous
import functools

_INTERPRET = False

import jax
import jax.numpy as jnp
from jax import lax
from jax.experimental import pallas as pl
from jax.experimental.pallas import tpu as pltpu
from jax.experimental.pallas import tpu_sc as plsc

N_NODES = 10000
D = 128

NC = 2
NS = 16
NW = NC * NS
N_PAD = 10240
ROWS_PER_TILE = N_PAD // NS
CHUNK = 128
E_PAD = 327680
CHUNKS_PER_W = E_PAD // (NW * CHUNK)
EDGES_PER_W = E_PAD // NW
HALF = CHUNKS_PER_W // 2


DEGW = 128


def _sc_degree_body(dst_hbm, out_hbm, didx, ones_rows, sem, acc):
    c = lax.axis_index("c")
    s = lax.axis_index("s")
    wid = c * NS + s

    ones16 = jnp.ones((16,), jnp.float32)
    zeros16 = jnp.zeros((16,), jnp.float32)

    nq = DEGW // 16

    def fill_z(i, _):
        ones_rows[i // nq, pl.ds((i % nq) * 16, 16)] = zeros16
        return 0

    lax.fori_loop(0, CHUNK * nq, fill_z, 0)

    row0 = s * ROWS_PER_TILE

    def zero_acc(j, _):
        pltpu.sync_copy(ones_rows, acc.at[pl.ds(row0 + j * CHUNK, CHUNK)])
        return 0

    lax.fori_loop(0, ROWS_PER_TILE // CHUNK, zero_acc, 0)

    def fill_o(i, _):
        ones_rows[i // nq, pl.ds((i % nq) * 16, 16)] = ones16
        return 0

    lax.fori_loop(0, CHUNK * nq, fill_o, 0)

    pltpu.sync_copy(dst_hbm.at[wid], didx)

    plsc.subcore_barrier()

    def fire(j, _):
        pltpu.async_copy(ones_rows, acc.at[didx.at[j]], sem, add=True)
        return 0

    lax.fori_loop(0, CHUNKS_PER_W, fire, 0)

    def drain(j, _):
        pltpu.make_async_copy(ones_rows, acc.at[didx.at[j]], sem).wait()
        return 0

    lax.fori_loop(0, CHUNKS_PER_W, drain, 0)

    plsc.subcore_barrier()

    pltpu.sync_copy(
        acc.at[pl.ds(row0, ROWS_PER_TILE)],
        out_hbm.at[c, pl.ds(row0, ROWS_PER_TILE)],
    )


def _sc_degree(dst_pad):
    mesh = plsc.VectorSubcoreMesh(core_axis_name="c", subcore_axis_name="s", num_cores=NC, num_subcores=NS)
    return pl.kernel(
        _sc_degree_body,
        out_type=jax.ShapeDtypeStruct((NC, N_PAD, DEGW), jnp.float32),
        mesh=mesh,
        interpret=_INTERPRET,
        scratch_types=[
            pltpu.VMEM((CHUNKS_PER_W, CHUNK), jnp.int32),
            pltpu.VMEM((CHUNK, DEGW), jnp.float32),
            pltpu.SemaphoreType.DMA,
            pltpu.VMEM_SHARED((N_PAD, DEGW), jnp.float32),
        ],
    )(dst_pad)


def _sc_agg_body(g_hbm, src_hbm, dst_hbm, out_hbm, sidx, didx, rows0, rows1, sem, acc):
    c = lax.axis_index("c")
    s = lax.axis_index("s")
    wid = c * NS + s

    zeros16 = jnp.zeros((16,), jnp.float32)

    def zero_rows(i, _):
        rows0[i // 8, pl.ds((i % 8) * 16, 16)] = zeros16
        return 0

    lax.fori_loop(0, CHUNK * 8, zero_rows, 0)

    row0 = s * ROWS_PER_TILE

    def zero_acc(j, _):
        pltpu.sync_copy(rows0, acc.at[pl.ds(row0 + j * CHUNK, CHUNK)])
        return 0

    lax.fori_loop(0, ROWS_PER_TILE // CHUNK, zero_acc, 0)

    plsc.subcore_barrier()

    def run_half(h):
        pltpu.sync_copy(src_hbm.at[wid, pl.ds(h * HALF, HALF)], sidx)
        pltpu.sync_copy(dst_hbm.at[wid, pl.ds(h * HALF, HALF)], didx)
        pltpu.async_copy(g_hbm.at[sidx.at[0]], rows0, sem)

        def body(i, _):
            j0 = 2 * i
            pltpu.async_copy(g_hbm.at[sidx.at[j0 + 1]], rows1, sem)
            pltpu.make_async_copy(g_hbm.at[sidx.at[j0]], rows0, sem).wait()
            pltpu.sync_copy(rows0, acc.at[didx.at[j0]], add=True)

            @pl.when(i < HALF // 2 - 1)
            def _():
                pltpu.async_copy(g_hbm.at[sidx.at[j0 + 2]], rows0, sem)

            pltpu.make_async_copy(g_hbm.at[sidx.at[j0 + 1]], rows1, sem).wait()
            pltpu.sync_copy(rows1, acc.at[didx.at[j0 + 1]], add=True)
            return 0

        lax.fori_loop(0, HALF // 2, body, 0)

    run_half(0)
    run_half(1)

    plsc.subcore_barrier()

    pltpu.sync_copy(
        acc.at[pl.ds(row0, ROWS_PER_TILE)],
        out_hbm.at[c, pl.ds(row0, ROWS_PER_TILE)],
    )


def _sc_agg(g, src_pad, dst_pad):
    mesh = plsc.VectorSubcoreMesh(core_axis_name="c", subcore_axis_name="s", num_cores=NC, num_subcores=NS)
    return pl.kernel(
        _sc_agg_body,
        out_type=jax.ShapeDtypeStruct((NC, N_PAD, D), jnp.float32),
        mesh=mesh,
        interpret=_INTERPRET,
        scratch_types=[
            pltpu.VMEM((HALF, CHUNK), jnp.int32),
            pltpu.VMEM((HALF, CHUNK), jnp.int32),
            pltpu.VMEM((CHUNK, D), jnp.float32),
            pltpu.VMEM((CHUNK, D), jnp.float32),
            pltpu.SemaphoreType.DMA,
            pltpu.VMEM_SHARED((N_PAD, D), jnp.float32),
        ],
    )(g, src_pad, dst_pad)


BLK = 400


def _tc_pre_body(x_ref, w_ref, degp_ref, g_ref, dinv_ref):
    i = pl.program_id(0)
    degs = degp_ref[0, pl.ds(i * BLK, BLK), :] + degp_ref[1, pl.ds(i * BLK, BLK), :]
    deg = degs[:, 0:1] + 1.0
    dinv = lax.rsqrt(deg)
    h = jnp.dot(x_ref[...], w_ref[...], preferred_element_type=jnp.float32)
    g_ref[...] = h * dinv
    dinv_ref[...] = dinv


def _tc_pre(x, W1, degp):
    return pl.pallas_call(
        _tc_pre_body,
        grid=(N_NODES // BLK,),
        interpret=_INTERPRET,
        in_specs=[
            pl.BlockSpec((BLK, D), lambda i: (i, 0)),
            pl.BlockSpec((D, D), lambda i: (0, 0)),
            pl.BlockSpec((NC, N_PAD, DEGW), lambda i: (0, 0, 0)),
        ],
        out_specs=[
            pl.BlockSpec((BLK, D), lambda i: (i, 0)),
            pl.BlockSpec((BLK, 1), lambda i: (i, 0)),
        ],
        out_shape=[
            jax.ShapeDtypeStruct((N_NODES, D), jnp.float32),
            jax.ShapeDtypeStruct((N_NODES, 1), jnp.float32),
        ],
    )(x, W1, degp)


def _tc_mid_body(p_ref, g1_ref, dinv_ref, b1_ref, w2_ref, g2_ref):
    s = p_ref[0] + p_ref[1] + g1_ref[...]
    dinv = dinv_ref[...]
    h = s * dinv + b1_ref[...]
    a = jnp.maximum(h, 0.0)
    h2 = jnp.dot(a, w2_ref[...], preferred_element_type=jnp.float32)
    g2_ref[...] = h2 * dinv


def _tc_mid(p, g1, dinv, b1, W2):
    return pl.pallas_call(
        _tc_mid_body,
        grid=(N_NODES // BLK,),
        interpret=_INTERPRET,
        in_specs=[
            pl.BlockSpec((NC, BLK, D), lambda i: (0, i, 0)),
            pl.BlockSpec((BLK, D), lambda i: (i, 0)),
            pl.BlockSpec((BLK, 1), lambda i: (i, 0)),
            pl.BlockSpec((1, D), lambda i: (0, 0)),
            pl.BlockSpec((D, D), lambda i: (0, 0)),
        ],
        out_specs=pl.BlockSpec((BLK, D), lambda i: (i, 0)),
        out_shape=jax.ShapeDtypeStruct((N_NODES, D), jnp.float32),
    )(p, g1, dinv, b1, W2)


def _tc_final_body(q_ref, g2_ref, dinv_ref, b2_ref, out_ref):
    s = q_ref[0] + q_ref[1] + g2_ref[...]
    z = s * dinv_ref[...] + b2_ref[...]
    m = jnp.max(z, axis=1, keepdims=True)
    e = jnp.exp(z - m)
    lse = m + jnp.log(jnp.sum(e, axis=1, keepdims=True))
    out_ref[...] = z - lse


def _tc_final(q, g2, dinv, b2):
    return pl.pallas_call(
        _tc_final_body,
        grid=(N_NODES // BLK,),
        interpret=_INTERPRET,
        in_specs=[
            pl.BlockSpec((NC, BLK, D), lambda i: (0, i, 0)),
            pl.BlockSpec((BLK, D), lambda i: (i, 0)),
            pl.BlockSpec((BLK, 1), lambda i: (i, 0)),
            pl.BlockSpec((1, D), lambda i: (0, 0)),
        ],
        out_specs=pl.BlockSpec((BLK, D), lambda i: (i, 0)),
        out_shape=jax.ShapeDtypeStruct((N_NODES, D), jnp.float32),
    )(q, g2, dinv, b2)


@jax.jit
def _run(x, edge_index, W1, b1, W2, b2):
    src = edge_index[0].astype(jnp.int32)
    dst = edge_index[1].astype(jnp.int32)
    n_edges = src.shape[0]
    pad = E_PAD - n_edges
    src_p = jnp.concatenate([src, jnp.zeros((pad,), jnp.int32)]).reshape(
        NW, CHUNKS_PER_W, CHUNK)
    dst_p = jnp.concatenate([dst, jnp.full((pad,), N_NODES, jnp.int32)]).reshape(
        NW, CHUNKS_PER_W, CHUNK)

    degp = _sc_degree(dst_p)
    g1, dinv = _tc_pre(x, W1, degp)
    p = _sc_agg(g1, src_p, dst_p)
    g2 = _tc_mid(p, g1, dinv, b1.reshape(1, D), W2)
    q = _sc_agg(g2, src_p, dst_p)
    return _tc_final(q, g2, dinv, b2.reshape(1, D))


def kernel(x, edge_index, W1, b1, W2, b2):
    return _run(x, edge_index, W1, b1, W2, b2)


# speedup vs baseline: 10.1667x; 1.2103x over previous
import functools

_INTERPRET = False

import jax
import jax.numpy as jnp
from jax import lax
from jax.experimental import pallas as pl
from jax.experimental.pallas import tpu as pltpu
from jax.experimental.pallas import tpu_sc as plsc

N_NODES = 10000
D = 128

NC = 2
NS = 16
NW = NC * NS
N_PAD = 10240
ROWS_PER_TILE = N_PAD // NS
CHUNK = 128
E_PAD = 327680
CHUNKS_PER_W = E_PAD // (NW * CHUNK)
EDGES_PER_W = E_PAD // NW
HALF = CHUNKS_PER_W // 2
TOTAL_CHUNKS = E_PAD // CHUNK
ROUND = 40
CH0 = 120
CH1 = 40


DEGW = 128


def _sc_degree_body(dst_hbm, out_hbm, didx, ones_rows, sem, acc):
    c = lax.axis_index("c")
    s = lax.axis_index("s")
    wid = c * NS + s

    ones16 = jnp.ones((16,), jnp.float32)
    zeros16 = jnp.zeros((16,), jnp.float32)

    nq = DEGW // 16

    def fill_z(i, _):
        ones_rows[i // nq, pl.ds((i % nq) * 16, 16)] = zeros16
        return 0

    lax.fori_loop(0, CHUNK * nq, fill_z, 0)

    row0 = s * ROWS_PER_TILE

    def zero_acc(j, _):
        pltpu.sync_copy(ones_rows, acc.at[pl.ds(row0 + j * CHUNK, CHUNK)])
        return 0

    lax.fori_loop(0, ROWS_PER_TILE // CHUNK, zero_acc, 0)

    def fill_o(i, _):
        ones_rows[i // nq, pl.ds((i % nq) * 16, 16)] = ones16
        return 0

    lax.fori_loop(0, CHUNK * nq, fill_o, 0)

    pltpu.sync_copy(dst_hbm.at[pl.ds(wid * CHUNKS_PER_W, CHUNKS_PER_W)], didx)

    plsc.subcore_barrier()

    def fire(j, _):
        pltpu.async_copy(ones_rows, acc.at[didx.at[j]], sem, add=True)
        return 0

    lax.fori_loop(0, CHUNKS_PER_W, fire, 0)

    def drain(j, _):
        pltpu.make_async_copy(ones_rows, acc.at[didx.at[j]], sem).wait()
        return 0

    lax.fori_loop(0, CHUNKS_PER_W, drain, 0)

    plsc.subcore_barrier()

    pltpu.sync_copy(
        acc.at[pl.ds(row0, ROWS_PER_TILE)],
        out_hbm.at[c, pl.ds(row0, ROWS_PER_TILE)],
    )


def _sc_degree(dst_pad):
    mesh = plsc.VectorSubcoreMesh(core_axis_name="c", subcore_axis_name="s", num_cores=NC, num_subcores=NS)
    return pl.kernel(
        _sc_degree_body,
        out_type=jax.ShapeDtypeStruct((NC, N_PAD, DEGW), jnp.float32),
        mesh=mesh,
        interpret=_INTERPRET,
        scratch_types=[
            pltpu.VMEM((CHUNKS_PER_W, CHUNK), jnp.int32),
            pltpu.VMEM((CHUNK, DEGW), jnp.float32),
            pltpu.SemaphoreType.DMA,
            pltpu.VMEM_SHARED((N_PAD, DEGW), jnp.float32),
        ],
    )(dst_pad)


def _sc_agg_body(g_hbm, src_hbm, dst_hbm, out_hbm, sidx, didx, rows0, rows1, sem, acc,
                 only_core=None):
    c = lax.axis_index("c")
    s = lax.axis_index("s")
    wid = c * NS + s

    zeros16 = jnp.zeros((16,), jnp.float32)

    def zero_rows(i, _):
        rows0[i // 8, pl.ds((i % 8) * 16, 16)] = zeros16
        return 0

    lax.fori_loop(0, CHUNK * 8, zero_rows, 0)

    row0 = s * ROWS_PER_TILE

    def zero_acc(j, _):
        pltpu.sync_copy(rows0, acc.at[pl.ds(row0 + j * CHUNK, CHUNK)])
        return 0

    lax.fori_loop(0, ROWS_PER_TILE // CHUNK, zero_acc, 0)

    plsc.subcore_barrier()

    def run_round(base):
        pltpu.sync_copy(src_hbm.at[pl.ds(base, ROUND)], sidx)
        pltpu.sync_copy(dst_hbm.at[pl.ds(base, ROUND)], didx)
        pltpu.async_copy(g_hbm.at[sidx.at[0]], rows0, sem)

        def body(i, _):
            j0 = 2 * i
            pltpu.async_copy(g_hbm.at[sidx.at[j0 + 1]], rows1, sem)
            pltpu.make_async_copy(g_hbm.at[sidx.at[j0]], rows0, sem).wait()
            pltpu.sync_copy(rows0, acc.at[didx.at[j0]], add=True)

            @pl.when(i < ROUND // 2 - 1)
            def _():
                pltpu.async_copy(g_hbm.at[sidx.at[j0 + 2]], rows0, sem)

            pltpu.make_async_copy(g_hbm.at[sidx.at[j0 + 1]], rows1, sem).wait()
            pltpu.sync_copy(rows1, acc.at[didx.at[j0 + 1]], add=True)
            return 0

        lax.fori_loop(0, ROUND // 2, body, 0)

    if only_core is None:
        @pl.when(c == 0)
        def _():
            for r in range(CH0 // ROUND):
                run_round(s * CH0 + r * ROUND)

        @pl.when(c == 1)
        def _():
            for r in range(CH1 // ROUND):
                run_round(NS * CH0 + s * CH1 + r * ROUND)
    else:
        @pl.when(c == only_core)
        def _():
            for r in range(2):
                run_round(wid * EDGES_PER_W // CHUNK + r * ROUND)

    plsc.subcore_barrier()

    pltpu.sync_copy(
        acc.at[pl.ds(row0, ROWS_PER_TILE)],
        out_hbm.at[c, pl.ds(row0, ROWS_PER_TILE)],
    )


def _sc_agg(g, src_pad, dst_pad):
    mesh = plsc.VectorSubcoreMesh(core_axis_name="c", subcore_axis_name="s", num_cores=NC, num_subcores=NS)
    return pl.kernel(
        _sc_agg_body,
        out_type=jax.ShapeDtypeStruct((NC, N_PAD, D), jnp.float32),
        mesh=mesh,
        interpret=_INTERPRET,
        scratch_types=[
            pltpu.VMEM((ROUND, CHUNK), jnp.int32),
            pltpu.VMEM((ROUND, CHUNK), jnp.int32),
            pltpu.VMEM((CHUNK, D), jnp.float32),
            pltpu.VMEM((CHUNK, D), jnp.float32),
            pltpu.SemaphoreType.DMA,
            pltpu.VMEM_SHARED((N_PAD, D), jnp.float32),
        ],
    )(g, src_pad, dst_pad)


def _sc_agg_only(g, src_pad, dst_pad, core):
    mesh = plsc.VectorSubcoreMesh(core_axis_name="c", subcore_axis_name="s", num_cores=NC, num_subcores=NS)
    return pl.kernel(
        functools.partial(_sc_agg_body, only_core=core),
        out_type=jax.ShapeDtypeStruct((NC, N_PAD, D), jnp.float32),
        mesh=mesh,
        interpret=_INTERPRET,
        scratch_types=[
            pltpu.VMEM((ROUND, CHUNK), jnp.int32),
            pltpu.VMEM((ROUND, CHUNK), jnp.int32),
            pltpu.VMEM((CHUNK, D), jnp.float32),
            pltpu.VMEM((CHUNK, D), jnp.float32),
            pltpu.SemaphoreType.DMA,
            pltpu.VMEM_SHARED((N_PAD, D), jnp.float32),
        ],
    )(g, src_pad, dst_pad)


BLK = 400


def _tc_pre_body(x_ref, w_ref, degp_ref, g_ref, dinv_ref):
    i = pl.program_id(0)
    degs = degp_ref[0, pl.ds(i * BLK, BLK), :] + degp_ref[1, pl.ds(i * BLK, BLK), :]
    deg = degs[:, 0:1] + 1.0
    dinv = lax.rsqrt(deg)
    h = jnp.dot(x_ref[...], w_ref[...], preferred_element_type=jnp.float32)
    g_ref[...] = h * dinv
    dinv_ref[...] = dinv


def _tc_pre(x, W1, degp):
    return pl.pallas_call(
        _tc_pre_body,
        grid=(N_NODES // BLK,),
        interpret=_INTERPRET,
        in_specs=[
            pl.BlockSpec((BLK, D), lambda i: (i, 0)),
            pl.BlockSpec((D, D), lambda i: (0, 0)),
            pl.BlockSpec((NC, N_PAD, DEGW), lambda i: (0, 0, 0)),
        ],
        out_specs=[
            pl.BlockSpec((BLK, D), lambda i: (i, 0)),
            pl.BlockSpec((BLK, 1), lambda i: (i, 0)),
        ],
        out_shape=[
            jax.ShapeDtypeStruct((N_NODES, D), jnp.float32),
            jax.ShapeDtypeStruct((N_NODES, 1), jnp.float32),
        ],
    )(x, W1, degp)


def _tc_mid_body(p_ref, g1_ref, dinv_ref, b1_ref, w2_ref, g2_ref):
    s = p_ref[0] + p_ref[1] + g1_ref[...]
    dinv = dinv_ref[...]
    h = s * dinv + b1_ref[...]
    a = jnp.maximum(h, 0.0)
    h2 = jnp.dot(a, w2_ref[...], preferred_element_type=jnp.float32)
    g2_ref[...] = h2 * dinv


def _tc_mid(p, g1, dinv, b1, W2):
    return pl.pallas_call(
        _tc_mid_body,
        grid=(N_NODES // BLK,),
        interpret=_INTERPRET,
        in_specs=[
            pl.BlockSpec((NC, BLK, D), lambda i: (0, i, 0)),
            pl.BlockSpec((BLK, D), lambda i: (i, 0)),
            pl.BlockSpec((BLK, 1), lambda i: (i, 0)),
            pl.BlockSpec((1, D), lambda i: (0, 0)),
            pl.BlockSpec((D, D), lambda i: (0, 0)),
        ],
        out_specs=pl.BlockSpec((BLK, D), lambda i: (i, 0)),
        out_shape=jax.ShapeDtypeStruct((N_NODES, D), jnp.float32),
    )(p, g1, dinv, b1, W2)


def _tc_final_body(q_ref, g2_ref, dinv_ref, b2_ref, out_ref):
    s = q_ref[0] + q_ref[1] + g2_ref[...]
    z = s * dinv_ref[...] + b2_ref[...]
    m = jnp.max(z, axis=1, keepdims=True)
    e = jnp.exp(z - m)
    lse = m + jnp.log(jnp.sum(e, axis=1, keepdims=True))
    out_ref[...] = z - lse


def _tc_final(q, g2, dinv, b2):
    return pl.pallas_call(
        _tc_final_body,
        grid=(N_NODES // BLK,),
        interpret=_INTERPRET,
        in_specs=[
            pl.BlockSpec((NC, BLK, D), lambda i: (0, i, 0)),
            pl.BlockSpec((BLK, D), lambda i: (i, 0)),
            pl.BlockSpec((BLK, 1), lambda i: (i, 0)),
            pl.BlockSpec((1, D), lambda i: (0, 0)),
        ],
        out_specs=pl.BlockSpec((BLK, D), lambda i: (i, 0)),
        out_shape=jax.ShapeDtypeStruct((N_NODES, D), jnp.float32),
    )(q, g2, dinv, b2)


@jax.jit
def _run(x, edge_index, W1, b1, W2, b2):
    src = edge_index[0].astype(jnp.int32)
    dst = edge_index[1].astype(jnp.int32)
    n_edges = src.shape[0]
    pad = E_PAD - n_edges
    src_p = jnp.concatenate([src, jnp.zeros((pad,), jnp.int32)]).reshape(
        TOTAL_CHUNKS, CHUNK)
    dst_p = jnp.concatenate([dst, jnp.full((pad,), N_NODES, jnp.int32)]).reshape(
        TOTAL_CHUNKS, CHUNK)

    degp = _sc_degree(dst_p)
    g1, dinv = _tc_pre(x, W1, degp)
    p = _sc_agg(g1, src_p, dst_p)
    g2 = _tc_mid(p, g1, dinv, b1.reshape(1, D), W2)
    q = _sc_agg(g2, src_p, dst_p)
    return _tc_final(q, g2, dinv, b2.reshape(1, D))


def kernel(x, edge_index, W1, b1, W2, b2):
    return _run(x, edge_index, W1, b1, W2, b2)


# speedup vs baseline: 10.7998x; 1.0623x over previous
import functools

_INTERPRET = False

import jax
import jax.numpy as jnp
from jax import lax
from jax.experimental import pallas as pl
from jax.experimental.pallas import tpu as pltpu
from jax.experimental.pallas import tpu_sc as plsc

N_NODES = 10000
D = 128

NC = 2
NS = 16
NW = NC * NS
N_PAD = 10240
ROWS_PER_TILE = N_PAD // NS
CHUNK = 128
E_PAD = 327680
CHUNKS_PER_W = E_PAD // (NW * CHUNK)
EDGES_PER_W = E_PAD // NW
HALF = CHUNKS_PER_W // 2
TOTAL_CHUNKS = E_PAD // CHUNK
ROUND = 16
CH0 = 144
CH1 = 16


DEGW = 128


def _sc_degree_body(dst_hbm, out_hbm, didx, ones_rows, sem, acc):
    c = lax.axis_index("c")
    s = lax.axis_index("s")
    wid = c * NS + s

    ones16 = jnp.ones((16,), jnp.float32)
    zeros16 = jnp.zeros((16,), jnp.float32)

    nq = DEGW // 16

    def fill_z(i, _):
        ones_rows[i // nq, pl.ds((i % nq) * 16, 16)] = zeros16
        return 0

    lax.fori_loop(0, CHUNK * nq, fill_z, 0)

    row0 = s * ROWS_PER_TILE

    def zero_acc(j, _):
        pltpu.sync_copy(ones_rows, acc.at[pl.ds(row0 + j * CHUNK, CHUNK)])
        return 0

    lax.fori_loop(0, ROWS_PER_TILE // CHUNK, zero_acc, 0)

    def fill_o(i, _):
        ones_rows[i // nq, pl.ds((i % nq) * 16, 16)] = ones16
        return 0

    lax.fori_loop(0, CHUNK * nq, fill_o, 0)

    pltpu.sync_copy(dst_hbm.at[pl.ds(wid * CHUNKS_PER_W, CHUNKS_PER_W)], didx)

    plsc.subcore_barrier()

    def fire(j, _):
        pltpu.async_copy(ones_rows, acc.at[didx.at[j]], sem, add=True)
        return 0

    lax.fori_loop(0, CHUNKS_PER_W, fire, 0)

    def drain(j, _):
        pltpu.make_async_copy(ones_rows, acc.at[didx.at[j]], sem).wait()
        return 0

    lax.fori_loop(0, CHUNKS_PER_W, drain, 0)

    plsc.subcore_barrier()

    pltpu.sync_copy(
        acc.at[pl.ds(row0, ROWS_PER_TILE)],
        out_hbm.at[c, pl.ds(row0, ROWS_PER_TILE)],
    )


def _sc_degree(dst_pad):
    mesh = plsc.VectorSubcoreMesh(core_axis_name="c", subcore_axis_name="s", num_cores=NC, num_subcores=NS)
    return pl.kernel(
        _sc_degree_body,
        out_type=jax.ShapeDtypeStruct((NC, N_PAD, DEGW), jnp.float32),
        mesh=mesh,
        interpret=_INTERPRET,
        scratch_types=[
            pltpu.VMEM((CHUNKS_PER_W, CHUNK), jnp.int32),
            pltpu.VMEM((CHUNK, DEGW), jnp.float32),
            pltpu.SemaphoreType.DMA,
            pltpu.VMEM_SHARED((N_PAD, DEGW), jnp.float32),
        ],
    )(dst_pad)


def _sc_agg_body(g_hbm, src_hbm, dst_hbm, out_hbm, sidx, didx, rows0, rows1, sem, acc,
                 only_core=None):
    c = lax.axis_index("c")
    s = lax.axis_index("s")
    wid = c * NS + s

    zeros16 = jnp.zeros((16,), jnp.float32)

    def zero_rows(i, _):
        rows0[i // 8, pl.ds((i % 8) * 16, 16)] = zeros16
        return 0

    lax.fori_loop(0, CHUNK * 8, zero_rows, 0)

    row0 = s * ROWS_PER_TILE

    def zero_acc(j, _):
        pltpu.sync_copy(rows0, acc.at[pl.ds(row0 + j * CHUNK, CHUNK)])
        return 0

    lax.fori_loop(0, ROWS_PER_TILE // CHUNK, zero_acc, 0)

    plsc.subcore_barrier()

    def run_round(base):
        pltpu.sync_copy(src_hbm.at[pl.ds(base, ROUND)], sidx)
        pltpu.sync_copy(dst_hbm.at[pl.ds(base, ROUND)], didx)
        pltpu.async_copy(g_hbm.at[sidx.at[0]], rows0, sem)

        def body(i, _):
            j0 = 2 * i
            pltpu.async_copy(g_hbm.at[sidx.at[j0 + 1]], rows1, sem)
            pltpu.make_async_copy(g_hbm.at[sidx.at[j0]], rows0, sem).wait()
            pltpu.sync_copy(rows0, acc.at[didx.at[j0]], add=True)

            @pl.when(i < ROUND // 2 - 1)
            def _():
                pltpu.async_copy(g_hbm.at[sidx.at[j0 + 2]], rows0, sem)

            pltpu.make_async_copy(g_hbm.at[sidx.at[j0 + 1]], rows1, sem).wait()
            pltpu.sync_copy(rows1, acc.at[didx.at[j0 + 1]], add=True)
            return 0

        lax.fori_loop(0, ROUND // 2, body, 0)

    if only_core is None:
        @pl.when(c == 0)
        def _():
            for r in range(CH0 // ROUND):
                run_round(s * CH0 + r * ROUND)

        @pl.when(c == 1)
        def _():
            for r in range(CH1 // ROUND):
                run_round(NS * CH0 + s * CH1 + r * ROUND)
    elif only_core == 0:
        @pl.when(c == 0)
        def _():
            for r in range(CH0 // ROUND):
                run_round(s * CH0 + r * ROUND)
    elif only_core == 1:
        @pl.when(c == 1)
        def _():
            for r in range(CH1 // ROUND):
                run_round(NS * CH0 + s * CH1 + r * ROUND)
    elif only_core == 2:
        pass
    elif only_core == 3:
        @pl.when(c == 1)
        def _():
            base = NS * CH0 + s * CH1
            pltpu.sync_copy(src_hbm.at[pl.ds(base, ROUND)], sidx)

            def body(i, _):
                pltpu.sync_copy(g_hbm.at[sidx.at[i]], rows0)
                return 0

            lax.fori_loop(0, ROUND, body, 0)
    elif only_core == 4:
        @pl.when(c == 1)
        def _():
            base = NS * CH0 + s * CH1
            pltpu.sync_copy(dst_hbm.at[pl.ds(base, ROUND)], didx)

            def body(i, _):
                pltpu.sync_copy(rows0, acc.at[didx.at[i]], add=True)
                return 0

            lax.fori_loop(0, ROUND, body, 0)

    plsc.subcore_barrier()

    pltpu.sync_copy(
        acc.at[pl.ds(row0, ROWS_PER_TILE)],
        out_hbm.at[c, pl.ds(row0, ROWS_PER_TILE)],
    )


def _sc_agg(g, src_pad, dst_pad):
    mesh = plsc.VectorSubcoreMesh(core_axis_name="c", subcore_axis_name="s", num_cores=NC, num_subcores=NS)
    return pl.kernel(
        _sc_agg_body,
        out_type=jax.ShapeDtypeStruct((NC, N_PAD, D), jnp.float32),
        mesh=mesh,
        interpret=_INTERPRET,
        scratch_types=[
            pltpu.VMEM((ROUND, CHUNK), jnp.int32),
            pltpu.VMEM((ROUND, CHUNK), jnp.int32),
            pltpu.VMEM((CHUNK, D), jnp.float32),
            pltpu.VMEM((CHUNK, D), jnp.float32),
            pltpu.SemaphoreType.DMA,
            pltpu.VMEM_SHARED((N_PAD, D), jnp.float32),
        ],
    )(g, src_pad, dst_pad)


def _sc_agg_only(g, src_pad, dst_pad, core):
    mesh = plsc.VectorSubcoreMesh(core_axis_name="c", subcore_axis_name="s", num_cores=NC, num_subcores=NS)
    return pl.kernel(
        functools.partial(_sc_agg_body, only_core=core),
        out_type=jax.ShapeDtypeStruct((NC, N_PAD, D), jnp.float32),
        mesh=mesh,
        interpret=_INTERPRET,
        scratch_types=[
            pltpu.VMEM((ROUND, CHUNK), jnp.int32),
            pltpu.VMEM((ROUND, CHUNK), jnp.int32),
            pltpu.VMEM((CHUNK, D), jnp.float32),
            pltpu.VMEM((CHUNK, D), jnp.float32),
            pltpu.SemaphoreType.DMA,
            pltpu.VMEM_SHARED((N_PAD, D), jnp.float32),
        ],
    )(g, src_pad, dst_pad)


BLK = 400


def _tc_pre_body(x_ref, w_ref, degp_ref, g_ref, dinv_ref):
    i = pl.program_id(0)
    degs = degp_ref[0, pl.ds(i * BLK, BLK), :] + degp_ref[1, pl.ds(i * BLK, BLK), :]
    deg = degs[:, 0:1] + 1.0
    dinv = lax.rsqrt(deg)
    h = jnp.dot(x_ref[...], w_ref[...], preferred_element_type=jnp.float32)
    g_ref[...] = h * dinv
    dinv_ref[...] = dinv


def _tc_pre(x, W1, degp):
    return pl.pallas_call(
        _tc_pre_body,
        grid=(N_NODES // BLK,),
        interpret=_INTERPRET,
        in_specs=[
            pl.BlockSpec((BLK, D), lambda i: (i, 0)),
            pl.BlockSpec((D, D), lambda i: (0, 0)),
            pl.BlockSpec((NC, N_PAD, DEGW), lambda i: (0, 0, 0)),
        ],
        out_specs=[
            pl.BlockSpec((BLK, D), lambda i: (i, 0)),
            pl.BlockSpec((BLK, 1), lambda i: (i, 0)),
        ],
        out_shape=[
            jax.ShapeDtypeStruct((N_NODES, D), jnp.float32),
            jax.ShapeDtypeStruct((N_NODES, 1), jnp.float32),
        ],
    )(x, W1, degp)


def _tc_mid_body(p_ref, g1_ref, dinv_ref, b1_ref, w2_ref, g2_ref):
    s = p_ref[0] + p_ref[1] + g1_ref[...]
    dinv = dinv_ref[...]
    h = s * dinv + b1_ref[...]
    a = jnp.maximum(h, 0.0)
    h2 = jnp.dot(a, w2_ref[...], preferred_element_type=jnp.float32)
    g2_ref[...] = h2 * dinv


def _tc_mid(p, g1, dinv, b1, W2):
    return pl.pallas_call(
        _tc_mid_body,
        grid=(N_NODES // BLK,),
        interpret=_INTERPRET,
        in_specs=[
            pl.BlockSpec((NC, BLK, D), lambda i: (0, i, 0)),
            pl.BlockSpec((BLK, D), lambda i: (i, 0)),
            pl.BlockSpec((BLK, 1), lambda i: (i, 0)),
            pl.BlockSpec((1, D), lambda i: (0, 0)),
            pl.BlockSpec((D, D), lambda i: (0, 0)),
        ],
        out_specs=pl.BlockSpec((BLK, D), lambda i: (i, 0)),
        out_shape=jax.ShapeDtypeStruct((N_NODES, D), jnp.float32),
    )(p, g1, dinv, b1, W2)


def _tc_final_body(q_ref, g2_ref, dinv_ref, b2_ref, out_ref):
    s = q_ref[0] + q_ref[1] + g2_ref[...]
    z = s * dinv_ref[...] + b2_ref[...]
    m = jnp.max(z, axis=1, keepdims=True)
    e = jnp.exp(z - m)
    lse = m + jnp.log(jnp.sum(e, axis=1, keepdims=True))
    out_ref[...] = z - lse


def _tc_final(q, g2, dinv, b2):
    return pl.pallas_call(
        _tc_final_body,
        grid=(N_NODES // BLK,),
        interpret=_INTERPRET,
        in_specs=[
            pl.BlockSpec((NC, BLK, D), lambda i: (0, i, 0)),
            pl.BlockSpec((BLK, D), lambda i: (i, 0)),
            pl.BlockSpec((BLK, 1), lambda i: (i, 0)),
            pl.BlockSpec((1, D), lambda i: (0, 0)),
        ],
        out_specs=pl.BlockSpec((BLK, D), lambda i: (i, 0)),
        out_shape=jax.ShapeDtypeStruct((N_NODES, D), jnp.float32),
    )(q, g2, dinv, b2)


@jax.jit
def _run(x, edge_index, W1, b1, W2, b2):
    src = edge_index[0].astype(jnp.int32)
    dst = edge_index[1].astype(jnp.int32)
    n_edges = src.shape[0]
    pad = E_PAD - n_edges
    src_p = jnp.concatenate([src, jnp.zeros((pad,), jnp.int32)]).reshape(
        TOTAL_CHUNKS, CHUNK)
    dst_p = jnp.concatenate([dst, jnp.full((pad,), N_NODES, jnp.int32)]).reshape(
        TOTAL_CHUNKS, CHUNK)

    degp = _sc_degree(dst_p)
    g1, dinv = _tc_pre(x, W1, degp)
    p = _sc_agg(g1, src_p, dst_p)
    g2 = _tc_mid(p, g1, dinv, b1.reshape(1, D), W2)
    q = _sc_agg(g2, src_p, dst_p)
    return _tc_final(q, g2, dinv, b2.reshape(1, D))


def kernel(x, edge_index, W1, b1, W2, b2):
    return _run(x, edge_index, W1, b1, W2, b2)
